# baseline (device time: 24872 ns/iter reference)
import os

import jax
import jax.numpy as jnp
from jax import lax
from jax.experimental import pallas as pl
from jax.experimental.pallas import tpu as pltpu

N_Z = 4
B, H, D, BS = 8, 8, 64, 16
NPAGES_LOCAL = 64
NKEYS = NPAGES_LOCAL * BS
NSLOTS = 64
HB = H * B

_ABLATE = os.environ.get("ABLATE", "none")

_DevT = getattr(pl, "DeviceIdType", None) or pltpu.DeviceIdType
_CParams = getattr(pltpu, "CompilerParams", None) or pltpu.TPUCompilerParams
_sem_signal = getattr(pl, "semaphore_signal", None) or pltpu.semaphore_signal
_sem_wait = getattr(pl, "semaphore_wait", None) or pltpu.semaphore_wait


def kernel(Q, K, V, bt, lens):
    lens2 = lens.reshape(B, 1)

    def body(q_ref, k_ref, v_ref, bt_ref, lens_ref, out_ref,
             kvm, vvm, mine, comm, kv_sems, send_sems, recv_sems):
        my_x = lax.axis_index("x")
        my_y = lax.axis_index("y")
        my_z = lax.axis_index("z")
        base = my_z * NPAGES_LOCAL

        kcopies = [pltpu.make_async_copy(k_ref.at[:, :, h, :], kvm.at[h],
                                         kv_sems.at[0, h]) for h in range(H)]
        vcopies = [pltpu.make_async_copy(v_ref.at[:, :, h, :], vvm.at[h],
                                         kv_sems.at[1, h]) for h in range(H)]
        if _ABLATE != "noattn":
            for c in kcopies:
                c.start()
            for c in vcopies:
                c.start()

        if _ABLATE != "nocomm":
            bar = pltpu.get_barrier_semaphore()
            for dz in (1, 2, 3):
                _sem_signal(bar, inc=1,
                            device_id=(my_x, my_y, (my_z + dz) % N_Z),
                            device_id_type=_DevT.MESH)
            _sem_wait(bar, 3)

        if _ABLATE == "noattn":
            mine[...] = jnp.full((3 * HB, D), 1.0, jnp.float32)
        else:
            ppk = base + lax.broadcasted_iota(jnp.int32, (B, NPAGES_LOCAL), 1)
            bt_val = bt_ref[...]
            lens_val = lens_ref[...]
            C = jnp.zeros((B, NPAGES_LOCAL), jnp.float32)
            for j in range(NSLOTS):
                btj = lax.slice(bt_val, (0, j), (B, j + 1))
                btj = jnp.where(lens_val > j, btj, -1)
                C = C + jnp.where(btj == ppk, 1.0, 0.0)

            rk = lax.broadcasted_iota(jnp.int32, (NPAGES_LOCAL, NKEYS), 0)
            ck = lax.broadcasted_iota(jnp.int32, (NPAGES_LOCAL, NKEYS), 1) >> 4
            Ekey = (rk == ck).astype(jnp.bfloat16)
            W = lax.dot_general(C.astype(jnp.bfloat16), Ekey,
                                (((1,), (0,)), ((), ())),
                                preferred_element_type=jnp.float32)

            scale = D ** -0.5
            qv = q_ref[...]
            for h in range(H):
                qh = (qv[:, 0, h, :] * scale).astype(jnp.bfloat16)
                kcopies[h].wait()
                kh = kvm[h].reshape(NKEYS, D).astype(jnp.bfloat16)
                s = lax.dot_general(qh, kh, (((1,), (1,)), ((), ())),
                                    preferred_element_type=jnp.float32)
                sm = jnp.where(W > 0, s, -1e30)
                m = jnp.max(sm, axis=1, keepdims=True)
                e = jnp.exp(sm - m) * W
                l = jnp.sum(e, axis=1, keepdims=True)
                vcopies[h].wait()
                vh = vvm[h].reshape(NKEYS, D).astype(jnp.bfloat16)
                num = lax.dot_general(e.astype(jnp.bfloat16), vh,
                                      (((1,), (0,)), ((), ())),
                                      preferred_element_type=jnp.float32)
                mine[h * B:(h + 1) * B, :] = num
                mine[HB + h * B:HB + (h + 1) * B, :] = \
                    jnp.broadcast_to(m, (B, D))
                mine[2 * HB + h * B:2 * HB + (h + 1) * B, :] = \
                    jnp.broadcast_to(l, (B, D))

        if _ABLATE != "nocomm":
            sends = []
            for dz in (1, 2, 3):
                r = pltpu.make_async_remote_copy(
                    src_ref=mine,
                    dst_ref=comm.at[my_z],
                    send_sem=send_sems.at[dz],
                    recv_sem=recv_sems.at[my_z],
                    device_id=(my_x, my_y, (my_z + dz) % N_Z),
                    device_id_type=_DevT.MESH,
                )
                r.start()
                sends.append(r)
            for dz in (1, 2, 3):
                src_z = (my_z + dz) % N_Z
                rr = pltpu.make_async_remote_copy(
                    src_ref=mine,
                    dst_ref=comm.at[src_z],
                    send_sem=send_sems.at[0],
                    recv_sem=recv_sems.at[src_z],
                    device_id=(my_x, my_y, src_z),
                    device_id_type=_DevT.MESH,
                )
                rr.wait_recv()
            for r in sends:
                r.wait_send()

        mine_val = mine[...]
        comm_val = comm[...]
        if _ABLATE == "nocomm":
            sel = [mine_val for _ in range(N_Z)]
        else:
            sel = [jnp.where(my_z == z, mine_val, comm_val[z])
                   for z in range(N_Z)]
        ms = [s_[HB:2 * HB, :] for s_ in sel]
        Mx = jnp.maximum(jnp.maximum(ms[0], ms[1]), jnp.maximum(ms[2], ms[3]))
        numsum = jnp.zeros((HB, D), jnp.float32)
        den = jnp.zeros((HB, D), jnp.float32)
        for z in range(N_Z):
            w = jnp.exp(ms[z] - Mx)
            numsum = numsum + w * sel[z][0:HB, :]
            den = den + w * sel[z][2 * HB:3 * HB, :]
        res = numsum / den

        rr_ = lax.broadcasted_iota(jnp.int32, (HB, HB), 0)
        cc_ = lax.broadcasted_iota(jnp.int32, (HB, HB), 1)
        P = (cc_ == ((rr_ & (B - 1)) * B + (rr_ >> 3))).astype(jnp.float32)
        res_bm = lax.dot_general(P, res, (((1,), (0,)), ((), ())),
                                 preferred_element_type=jnp.float32)
        out_ref[...] = res_bm.reshape(B, 1, H, D)

    out = pl.pallas_call(
        body,
        out_shape=jax.ShapeDtypeStruct((B, 1, H, D), jnp.float32),
        in_specs=[
            pl.BlockSpec(memory_space=pltpu.VMEM),
            pl.BlockSpec(memory_space=pltpu.MemorySpace.HBM),
            pl.BlockSpec(memory_space=pltpu.MemorySpace.HBM),
            pl.BlockSpec(memory_space=pltpu.VMEM),
            pl.BlockSpec(memory_space=pltpu.VMEM),
        ],
        out_specs=pl.BlockSpec(memory_space=pltpu.VMEM),
        scratch_shapes=[
            pltpu.VMEM((H, NPAGES_LOCAL, BS, D), jnp.float32),
            pltpu.VMEM((H, NPAGES_LOCAL, BS, D), jnp.float32),
            pltpu.VMEM((3 * HB, D), jnp.float32),
            pltpu.VMEM((N_Z, 3 * HB, D), jnp.float32),
            pltpu.SemaphoreType.DMA((2, H)),
            pltpu.SemaphoreType.DMA((N_Z,)),
            pltpu.SemaphoreType.DMA((N_Z,)),
        ],
        compiler_params=(_CParams() if _ABLATE == "nocomm"
                         else _CParams(collective_id=0)),
    )(Q, K, V, bt, lens2)
    return out


# device time: 16497 ns/iter; 1.5077x vs baseline; 1.5077x over previous
import os

import jax
import jax.numpy as jnp
from jax import lax
from jax.experimental import pallas as pl
from jax.experimental.pallas import tpu as pltpu

N_Z = 4
B, H, D, BS = 8, 8, 64, 16
NPAGES_LOCAL = 64
NKEYS = NPAGES_LOCAL * BS
NSLOTS = 64
HB = H * B
HD = H * D

_ABLATE = os.environ.get("ABLATE", "none")

_DevT = getattr(pl, "DeviceIdType", None) or pltpu.DeviceIdType
_CParams = getattr(pltpu, "CompilerParams", None) or pltpu.TPUCompilerParams
_sem_signal = getattr(pl, "semaphore_signal", None) or pltpu.semaphore_signal
_sem_wait = getattr(pl, "semaphore_wait", None) or pltpu.semaphore_wait


def kernel(Q, K, V, bt, lens):
    Kr = K.reshape(NKEYS, HD).astype(jnp.bfloat16)
    Vr = V.reshape(NKEYS, HD).astype(jnp.bfloat16)
    lens2 = lens.reshape(B, 1)

    def body(q_ref, k_ref, v_ref, bt_ref, lens_ref, out_ref,
             qexp, mine, comm, send_sems, recv_sems):
        my_x = lax.axis_index("x")
        my_y = lax.axis_index("y")
        my_z = lax.axis_index("z")
        base = my_z * NPAGES_LOCAL

        if _ABLATE != "nocomm":
            bar = pltpu.get_barrier_semaphore()
            for dz in (1, 2, 3):
                _sem_signal(bar, inc=1,
                            device_id=(my_x, my_y, (my_z + dz) % N_Z),
                            device_id_type=_DevT.MESH)
            _sem_wait(bar, 3)

        if _ABLATE == "noattn":
            mine[...] = jnp.full((3 * HB, D), 1.0, jnp.float32)
        else:
            ppk = base + lax.broadcasted_iota(jnp.int32, (B, NPAGES_LOCAL), 1)
            bt_val = bt_ref[...]
            lens_val = lens_ref[...]
            C = jnp.zeros((B, NPAGES_LOCAL), jnp.float32)
            for j in range(NSLOTS):
                btj = lax.slice(bt_val, (0, j), (B, j + 1))
                btj = jnp.where(lens_val > j, btj, -1)
                C = C + jnp.where(btj == ppk, 1.0, 0.0)

            rk = lax.broadcasted_iota(jnp.int32, (NPAGES_LOCAL, NKEYS), 0)
            ck = lax.broadcasted_iota(jnp.int32, (NPAGES_LOCAL, NKEYS), 1) >> 4
            Ekey = (rk == ck).astype(jnp.bfloat16)
            Wk = lax.dot_general(C.astype(jnp.bfloat16), Ekey,
                                 (((1,), (0,)), ((), ())),
                                 preferred_element_type=jnp.float32)
            rb = lax.broadcasted_iota(jnp.int32, (HB, B), 0) & (B - 1)
            cb = lax.broadcasted_iota(jnp.int32, (HB, B), 1)
            Erow = (rb == cb).astype(jnp.bfloat16)
            W = lax.dot_general(Erow, Wk.astype(jnp.bfloat16),
                                (((1,), (0,)), ((), ())),
                                preferred_element_type=jnp.float32)

            scale = D ** -0.5
            qexp[...] = jnp.zeros((HB, HD), jnp.float32)
            qv = q_ref[...]
            for h in range(H):
                qexp[h * B:(h + 1) * B, h * D:(h + 1) * D] = \
                    qv[:, 0, h, :] * scale
            qe = qexp[...].astype(jnp.bfloat16)

            s = lax.dot_general(qe, k_ref[...], (((1,), (1,)), ((), ())),
                                preferred_element_type=jnp.float32)
            sm = jnp.where(W > 0, s, -1e30)
            m = jnp.max(sm, axis=1, keepdims=True)
            e = jnp.exp(sm - m) * W
            l = jnp.sum(e, axis=1, keepdims=True)
            num_full = lax.dot_general(e.astype(jnp.bfloat16), v_ref[...],
                                       (((1,), (0,)), ((), ())),
                                       preferred_element_type=jnp.float32)

            for h in range(H):
                mine[h * B:(h + 1) * B, :] = \
                    num_full[h * B:(h + 1) * B, h * D:(h + 1) * D]
            mine[HB:2 * HB, :] = jnp.broadcast_to(m, (HB, D))
            mine[2 * HB:3 * HB, :] = jnp.broadcast_to(l, (HB, D))

        if _ABLATE != "nocomm":
            sends = []
            for dz in (1, 2, 3):
                r = pltpu.make_async_remote_copy(
                    src_ref=mine,
                    dst_ref=comm.at[my_z],
                    send_sem=send_sems.at[dz],
                    recv_sem=recv_sems.at[my_z],
                    device_id=(my_x, my_y, (my_z + dz) % N_Z),
                    device_id_type=_DevT.MESH,
                )
                r.start()
                sends.append(r)
            for dz in (1, 2, 3):
                src_z = (my_z + dz) % N_Z
                rr = pltpu.make_async_remote_copy(
                    src_ref=mine,
                    dst_ref=comm.at[src_z],
                    send_sem=send_sems.at[0],
                    recv_sem=recv_sems.at[src_z],
                    device_id=(my_x, my_y, src_z),
                    device_id_type=_DevT.MESH,
                )
                rr.wait_recv()
            for r in sends:
                r.wait_send()

        mine_val = mine[...]
        comm_val = comm[...]
        if _ABLATE == "nocomm":
            sel = [mine_val for _ in range(N_Z)]
        else:
            sel = [jnp.where(my_z == z, mine_val, comm_val[z])
                   for z in range(N_Z)]
        ms = [s_[HB:2 * HB, :] for s_ in sel]
        Mx = jnp.maximum(jnp.maximum(ms[0], ms[1]), jnp.maximum(ms[2], ms[3]))
        numsum = jnp.zeros((HB, D), jnp.float32)
        den = jnp.zeros((HB, D), jnp.float32)
        for z in range(N_Z):
            w = jnp.exp(ms[z] - Mx)
            numsum = numsum + w * sel[z][0:HB, :]
            den = den + w * sel[z][2 * HB:3 * HB, :]
        res = numsum / den

        rr_ = lax.broadcasted_iota(jnp.int32, (HB, HB), 0)
        cc_ = lax.broadcasted_iota(jnp.int32, (HB, HB), 1)
        P = (cc_ == ((rr_ & (B - 1)) * B + (rr_ >> 3))).astype(jnp.float32)
        res_bm = lax.dot_general(P, res, (((1,), (0,)), ((), ())),
                                 preferred_element_type=jnp.float32)
        out_ref[...] = res_bm.reshape(B, 1, H, D)

    out = pl.pallas_call(
        body,
        out_shape=jax.ShapeDtypeStruct((B, 1, H, D), jnp.float32),
        in_specs=[pl.BlockSpec(memory_space=pltpu.VMEM)] * 5,
        out_specs=pl.BlockSpec(memory_space=pltpu.VMEM),
        scratch_shapes=[
            pltpu.VMEM((HB, HD), jnp.float32),
            pltpu.VMEM((3 * HB, D), jnp.float32),
            pltpu.VMEM((N_Z, 3 * HB, D), jnp.float32),
            pltpu.SemaphoreType.DMA((N_Z,)),
            pltpu.SemaphoreType.DMA((N_Z,)),
        ],
        compiler_params=(_CParams() if _ABLATE == "nocomm"
                         else _CParams(collective_id=0)),
    )(Q, Kr, Vr, bt, lens2)
    return out


# device time: 13935 ns/iter; 1.7849x vs baseline; 1.1839x over previous
import jax
import jax.numpy as jnp
from jax import lax
from jax.experimental import pallas as pl
from jax.experimental.pallas import tpu as pltpu

N_Z = 4
B, H, D, BS = 8, 8, 64, 16
NPAGES_LOCAL = 64
NKEYS = NPAGES_LOCAL * BS
NSLOTS = 64
HB = H * B
HD = H * D

_DevT = getattr(pl, "DeviceIdType", None) or pltpu.DeviceIdType
_CParams = getattr(pltpu, "CompilerParams", None) or pltpu.TPUCompilerParams
_sem_signal = getattr(pl, "semaphore_signal", None) or pltpu.semaphore_signal
_sem_wait = getattr(pl, "semaphore_wait", None) or pltpu.semaphore_wait


def kernel(Q, K, V, bt, lens):
    Kr = K.reshape(NKEYS, HD).astype(jnp.bfloat16)
    Vr = V.reshape(NKEYS, HD).astype(jnp.bfloat16)
    lens2 = lens.reshape(B, 1)

    def body(q_ref, k_ref, v_ref, bt_ref, lens_ref, out_ref,
             qexp, mine, comm, send_sems, recv_sems):
        my_x = lax.axis_index("x")
        my_y = lax.axis_index("y")
        my_z = lax.axis_index("z")
        base = my_z * NPAGES_LOCAL

        bar = pltpu.get_barrier_semaphore()
        for dz in (1, 2, 3):
            _sem_signal(bar, inc=1,
                        device_id=(my_x, my_y, (my_z + dz) % N_Z),
                        device_id_type=_DevT.MESH)

        bt3 = bt_ref[...].reshape(B, NSLOTS, 1)
        page3 = base + lax.broadcasted_iota(
            jnp.int32, (B, NSLOTS, NPAGES_LOCAL), 2)
        slot3 = lax.broadcasted_iota(
            jnp.int32, (B, NSLOTS, NPAGES_LOCAL), 1)
        lens3 = lens_ref[...].reshape(B, 1, 1)
        hit = jnp.where((bt3 == page3) & (slot3 < lens3), 1.0, 0.0)
        C = jnp.sum(hit, axis=1)

        rk = lax.broadcasted_iota(jnp.int32, (NPAGES_LOCAL, NKEYS), 0)
        ck = lax.broadcasted_iota(jnp.int32, (NPAGES_LOCAL, NKEYS), 1) >> 4
        Ekey = (rk == ck).astype(jnp.bfloat16)
        Wk = lax.dot_general(C.astype(jnp.bfloat16), Ekey,
                             (((1,), (0,)), ((), ())),
                             preferred_element_type=jnp.float32)
        rb = lax.broadcasted_iota(jnp.int32, (HB, B), 0) & (B - 1)
        cb = lax.broadcasted_iota(jnp.int32, (HB, B), 1)
        Erow = (rb == cb).astype(jnp.bfloat16)
        W = lax.dot_general(Erow, Wk.astype(jnp.bfloat16),
                            (((1,), (0,)), ((), ())),
                            preferred_element_type=jnp.float32)

        scale = D ** -0.5
        qexp[...] = jnp.zeros((HB, HD), jnp.float32)
        qv = q_ref[...]
        for h in range(H):
            qexp[h * B:(h + 1) * B, h * D:(h + 1) * D] = \
                qv[:, 0, h, :] * scale
        qe = qexp[...].astype(jnp.bfloat16)

        s = lax.dot_general(qe, k_ref[...], (((1,), (1,)), ((), ())),
                            preferred_element_type=jnp.float32)
        sm = jnp.where(W > 0, s, -1e30)
        m = jnp.max(sm, axis=1, keepdims=True)
        e = jnp.exp(sm - m) * W
        l = jnp.sum(e, axis=1, keepdims=True)
        num_full = lax.dot_general(e.astype(jnp.bfloat16), v_ref[...],
                                   (((1,), (0,)), ((), ())),
                                   preferred_element_type=jnp.float32)

        for h in range(H):
            mine[h * B:(h + 1) * B, :] = \
                num_full[h * B:(h + 1) * B, h * D:(h + 1) * D]
        lane = lax.broadcasted_iota(jnp.int32, (HB, D), 1)
        mine[HB:2 * HB, :] = jnp.where(
            lane < D // 2,
            jnp.broadcast_to(m, (HB, D)),
            jnp.broadcast_to(l, (HB, D)))

        _sem_wait(bar, 3)
        sends = []
        for dz in (1, 2, 3):
            r = pltpu.make_async_remote_copy(
                src_ref=mine,
                dst_ref=comm.at[my_z],
                send_sem=send_sems.at[dz],
                recv_sem=recv_sems.at[my_z],
                device_id=(my_x, my_y, (my_z + dz) % N_Z),
                device_id_type=_DevT.MESH,
            )
            r.start()
            sends.append(r)
        for dz in (1, 2, 3):
            src_z = (my_z + dz) % N_Z
            rr = pltpu.make_async_remote_copy(
                src_ref=mine,
                dst_ref=comm.at[src_z],
                send_sem=send_sems.at[0],
                recv_sem=recv_sems.at[src_z],
                device_id=(my_x, my_y, src_z),
                device_id_type=_DevT.MESH,
            )
            rr.wait_recv()
        for r in sends:
            r.wait_send()

        mine_val = mine[...]
        comm_val = comm[...]
        sel = [jnp.where(my_z == z, mine_val, comm_val[z])
               for z in range(N_Z)]
        ms = [s_[HB:2 * HB, 0:1] for s_ in sel]
        ls = [s_[HB:2 * HB, D // 2:D // 2 + 1] for s_ in sel]
        Mx = jnp.maximum(jnp.maximum(ms[0], ms[1]), jnp.maximum(ms[2], ms[3]))
        numsum = jnp.zeros((HB, D), jnp.float32)
        den = jnp.zeros((HB, 1), jnp.float32)
        for z in range(N_Z):
            w = jnp.exp(ms[z] - Mx)
            numsum = numsum + w * sel[z][0:HB, :]
            den = den + w * ls[z]
        res = numsum / den

        rr_ = lax.broadcasted_iota(jnp.int32, (HB, HB), 0)
        cc_ = lax.broadcasted_iota(jnp.int32, (HB, HB), 1)
        P = (cc_ == ((rr_ & (B - 1)) * B + (rr_ >> 3))).astype(jnp.float32)
        res_bm = lax.dot_general(P, res, (((1,), (0,)), ((), ())),
                                 preferred_element_type=jnp.float32)
        out_ref[...] = res_bm.reshape(B, 1, H, D)

    out = pl.pallas_call(
        body,
        out_shape=jax.ShapeDtypeStruct((B, 1, H, D), jnp.float32),
        in_specs=[pl.BlockSpec(memory_space=pltpu.VMEM)] * 5,
        out_specs=pl.BlockSpec(memory_space=pltpu.VMEM),
        scratch_shapes=[
            pltpu.VMEM((HB, HD), jnp.float32),
            pltpu.VMEM((2 * HB, D), jnp.float32),
            pltpu.VMEM((N_Z, 2 * HB, D), jnp.float32),
            pltpu.SemaphoreType.DMA((N_Z,)),
            pltpu.SemaphoreType.DMA((N_Z,)),
        ],
        compiler_params=_CParams(collective_id=0),
    )(Q, Kr, Vr, bt, lens2)
    return out
